# baseline (speedup 1.0000x reference)
import functools

import jax
import jax.numpy as jnp
from jax import lax
from jax.experimental import pallas as pl
from jax.experimental.pallas import tpu as pltpu
from jax.experimental.pallas import tpu_sc as plsc

N = 18432
K = 1024
D = 64
TN = 512
GRID = N // TN
COMMITMENT_COST = 0.25
LOSS_SCALE = (1.0 + COMMITMENT_COST) / (N * D)

_SC_INFO = plsc.get_sparse_core_info()
NC = _SC_INFO.num_cores
NS = _SC_INFO.num_subcores
NW = NC * NS
BPW = N // NW
CHUNK = 96
NCHUNK = BPW // CHUNK


def _dist_argmin_body(zt_ref, wt_ref, idx_ref, loss_ref):
    i = pl.program_id(0)
    zt = zt_ref[...]
    wt = wt_ref[...]
    zsq = jnp.sum(zt * zt, axis=0, keepdims=True)
    wsq = jnp.sum(wt * wt, axis=0, keepdims=True)
    m = jax.lax.dot_general(wt, zt, (((0,), (0,)), ((), ())),
                            preferred_element_type=jnp.float32)
    d = (zsq + wsq.reshape(K, 1)) - 2.0 * m
    dmin = jnp.min(d, axis=0, keepdims=True)
    iota = jax.lax.broadcasted_iota(jnp.int32, d.shape, 0).astype(jnp.float32)
    idxf = jnp.min(jnp.where(d == dmin, iota, float(K)), axis=0)
    idx_ref[...] = idxf.astype(jnp.int32)[None, None, :]
    part = jnp.sum(dmin, keepdims=True)

    @pl.when(i == 0)
    def _init():
        loss_ref[...] = jnp.zeros_like(loss_ref)

    loss_ref[...] += part

    @pl.when(i == GRID - 1)
    def _final():
        loss_ref[...] = loss_ref[...] * LOSS_SCALE


_dist_argmin = pl.pallas_call(
    _dist_argmin_body,
    grid=(GRID,),
    in_specs=[
        pl.BlockSpec((D, TN), lambda i: (0, i)),
        pl.BlockSpec((D, K), lambda i: (0, 0)),
    ],
    out_specs=(
        pl.BlockSpec((1, 1, TN), lambda i: (i, 0, 0)),
        pl.BlockSpec((1, 1), lambda i: (0, 0)),
    ),
    out_shape=(
        jax.ShapeDtypeStruct((GRID, 1, TN), jnp.int32),
        jax.ShapeDtypeStruct((1, 1), jnp.float32),
    ),
)


@functools.partial(
    pl.kernel,
    mesh=plsc.VectorSubcoreMesh(core_axis_name="c", subcore_axis_name="s"),
    compiler_params=pltpu.CompilerParams(use_tc_tiling_on_sc=False),
    out_type=jax.ShapeDtypeStruct((N, D), jnp.float32),
    scratch_types=[
        pltpu.VMEM((NCHUNK, CHUNK), jnp.int32),
        pltpu.VMEM((BPW, D), jnp.float32),
        pltpu.SemaphoreType.DMA,
    ],
)
def _sc_gather(idx_hbm, w_hbm, out_hbm, idx_v, rows_v, sem):
    wid = lax.axis_index("s") * NC + lax.axis_index("c")
    pltpu.sync_copy(idx_hbm.at[wid], idx_v)
    copies = []
    for j in range(NCHUNK):
        copies.append(pltpu.async_copy(
            w_hbm.at[idx_v.at[j]],
            rows_v.at[pl.ds(j * CHUNK, CHUNK)],
            sem,
        ))
    for c in copies:
        c.wait()
    pltpu.sync_copy(rows_v, out_hbm.at[pl.ds(wid * BPW, BPW)])


def _transpose_body(zq_ref, out_ref):
    out_ref[...] = zq_ref[...].T


_transpose_ep = pl.pallas_call(
    _transpose_body,
    grid=(GRID,),
    in_specs=[pl.BlockSpec((TN, D), lambda i: (i, 0))],
    out_specs=pl.BlockSpec((D, TN), lambda i: (0, i)),
    out_shape=jax.ShapeDtypeStruct((D, N), jnp.float32),
)


def kernel(z, W):
    idx3, loss = _dist_argmin(z.T, W.T)
    idx = idx3.reshape(NW, NCHUNK, CHUNK)
    zq = _sc_gather(idx, W)
    zqt = _transpose_ep(zq)
    return zqt.T, loss[0, 0]


# speedup vs baseline: 1.3349x; 1.3349x over previous
import functools

import jax
import jax.numpy as jnp
from jax import lax
from jax.experimental import pallas as pl
from jax.experimental.pallas import tpu as pltpu
from jax.experimental.pallas import tpu_sc as plsc

N = 18432
K = 1024
D = 64
TN = 2304
GRID = N // TN
COMMITMENT_COST = 0.25
LOSS_SCALE = (1.0 + COMMITMENT_COST) / (N * D)

_SC_INFO = plsc.get_sparse_core_info()
NC = _SC_INFO.num_cores
NS = _SC_INFO.num_subcores
NW = NC * NS
BPW = N // NW
CHUNK = 96
NCHUNK = BPW // CHUNK


def _dist_argmin_body(zt_ref, wt_ref, idx_ref, loss_ref):
    i = pl.program_id(0)
    zt = zt_ref[...]
    wt = wt_ref[...]
    zsq = jnp.sum(zt * zt, axis=0, keepdims=True)
    wsq = jnp.sum(wt * wt, axis=0, keepdims=True)
    m = jax.lax.dot_general(wt, zt, (((0,), (0,)), ((), ())),
                            preferred_element_type=jnp.float32)
    d = (zsq + wsq.reshape(K, 1)) - 2.0 * m
    dmin = jnp.min(d, axis=0, keepdims=True)
    iota = jax.lax.broadcasted_iota(jnp.int32, d.shape, 0).astype(jnp.float32)
    idxf = jnp.min(jnp.where(d == dmin, iota, float(K)), axis=0)
    idx_ref[...] = idxf.astype(jnp.int32)[None, None, :]
    part = jnp.sum(dmin, keepdims=True)

    @pl.when(i == 0)
    def _init():
        loss_ref[...] = jnp.zeros_like(loss_ref)

    loss_ref[...] += part

    @pl.when(i == GRID - 1)
    def _final():
        loss_ref[...] = loss_ref[...] * LOSS_SCALE


_dist_argmin = pl.pallas_call(
    _dist_argmin_body,
    grid=(GRID,),
    in_specs=[
        pl.BlockSpec((D, TN), lambda i: (0, i)),
        pl.BlockSpec((D, K), lambda i: (0, 0)),
    ],
    out_specs=(
        pl.BlockSpec((1, 1, TN), lambda i: (i, 0, 0)),
        pl.BlockSpec((1, 1), lambda i: (0, 0)),
    ),
    out_shape=(
        jax.ShapeDtypeStruct((GRID, 1, TN), jnp.int32),
        jax.ShapeDtypeStruct((1, 1), jnp.float32),
    ),
)


@functools.partial(
    pl.kernel,
    mesh=plsc.VectorSubcoreMesh(core_axis_name="c", subcore_axis_name="s"),
    compiler_params=pltpu.CompilerParams(use_tc_tiling_on_sc=False),
    out_type=jax.ShapeDtypeStruct((N, D), jnp.float32),
    scratch_types=[
        pltpu.VMEM((NCHUNK, CHUNK), jnp.int32),
        pltpu.VMEM((BPW, D), jnp.float32),
        pltpu.SemaphoreType.DMA,
    ],
)
def _sc_gather(idx_hbm, w_hbm, out_hbm, idx_v, rows_v, sem):
    wid = lax.axis_index("s") * NC + lax.axis_index("c")
    pltpu.sync_copy(idx_hbm.at[wid], idx_v)
    copies = []
    for j in range(NCHUNK):
        copies.append(pltpu.async_copy(
            w_hbm.at[idx_v.at[j]],
            rows_v.at[pl.ds(j * CHUNK, CHUNK)],
            sem,
        ))
    for c in copies:
        c.wait()
    pltpu.sync_copy(rows_v, out_hbm.at[pl.ds(wid * BPW, BPW)])


def kernel(z, W):
    idx3, loss = _dist_argmin(z.T, W.T)
    idx = idx3.reshape(NW, NCHUNK, CHUNK)
    zq = _sc_gather(idx, W)
    return zq, loss[0, 0]


# speedup vs baseline: 1.3535x; 1.0139x over previous
import functools

import jax
import jax.numpy as jnp
from jax import lax
from jax.experimental import pallas as pl
from jax.experimental.pallas import tpu as pltpu
from jax.experimental.pallas import tpu_sc as plsc

N = 18432
K = 1024
D = 64
TN = 4608
GRID = N // TN
COMMITMENT_COST = 0.25
LOSS_SCALE = (1.0 + COMMITMENT_COST) / (N * D)

_SC_INFO = plsc.get_sparse_core_info()
NC = _SC_INFO.num_cores
NS = _SC_INFO.num_subcores
NW = NC * NS
BPW = N // NW
CHUNK = 96
NCHUNK = BPW // CHUNK


def _dist_argmin_body(zt_ref, wt_ref, idx_ref, loss_ref):
    i = pl.program_id(0)
    zt = zt_ref[...]
    wt = wt_ref[...]
    zsq = jnp.sum(zt * zt, axis=0, keepdims=True)
    wsq = jnp.sum(wt * wt, axis=0, keepdims=True)
    m = jax.lax.dot_general(wt, zt, (((0,), (0,)), ((), ())),
                            preferred_element_type=jnp.float32)
    d = (zsq + wsq.reshape(K, 1)) - 2.0 * m
    dmin = jnp.min(d, axis=0, keepdims=True)
    iota = jax.lax.broadcasted_iota(jnp.int32, d.shape, 0).astype(jnp.float32)
    idxf = jnp.min(jnp.where(d == dmin, iota, float(K)), axis=0)
    idx_ref[...] = idxf.astype(jnp.int32)[None, None, :]
    part = jnp.sum(dmin, keepdims=True)

    @pl.when(i == 0)
    def _init():
        loss_ref[...] = jnp.zeros_like(loss_ref)

    loss_ref[...] += part

    @pl.when(i == GRID - 1)
    def _final():
        loss_ref[...] = loss_ref[...] * LOSS_SCALE


_dist_argmin = pl.pallas_call(
    _dist_argmin_body,
    grid=(GRID,),
    in_specs=[
        pl.BlockSpec((D, TN), lambda i: (0, i)),
        pl.BlockSpec((D, K), lambda i: (0, 0)),
    ],
    out_specs=(
        pl.BlockSpec((1, 1, TN), lambda i: (i, 0, 0)),
        pl.BlockSpec((1, 1), lambda i: (0, 0)),
    ),
    out_shape=(
        jax.ShapeDtypeStruct((GRID, 1, TN), jnp.int32),
        jax.ShapeDtypeStruct((1, 1), jnp.float32),
    ),
)


@functools.partial(
    pl.kernel,
    mesh=plsc.VectorSubcoreMesh(core_axis_name="c", subcore_axis_name="s"),
    compiler_params=pltpu.CompilerParams(use_tc_tiling_on_sc=False),
    out_type=jax.ShapeDtypeStruct((N, D), jnp.float32),
    scratch_types=[
        pltpu.VMEM((NCHUNK, CHUNK), jnp.int32),
        pltpu.VMEM((BPW, D), jnp.float32),
        pltpu.SemaphoreType.DMA,
    ],
)
def _sc_gather(idx_hbm, w_hbm, out_hbm, idx_v, rows_v, sem):
    wid = lax.axis_index("s") * NC + lax.axis_index("c")
    pltpu.sync_copy(idx_hbm.at[wid], idx_v)
    copies = []
    for j in range(NCHUNK):
        copies.append(pltpu.async_copy(
            w_hbm.at[idx_v.at[j]],
            rows_v.at[pl.ds(j * CHUNK, CHUNK)],
            sem,
        ))
    for c in copies:
        c.wait()
    pltpu.sync_copy(rows_v, out_hbm.at[pl.ds(wid * BPW, BPW)])


def kernel(z, W):
    idx3, loss = _dist_argmin(z.T, W.T)
    idx = idx3.reshape(NW, NCHUNK, CHUNK)
    zq = _sc_gather(idx, W)
    return zq, loss[0, 0]


# speedup vs baseline: 1.3624x; 1.0066x over previous
import functools

import jax
import jax.numpy as jnp
from jax import lax
from jax.experimental import pallas as pl
from jax.experimental.pallas import tpu as pltpu
from jax.experimental.pallas import tpu_sc as plsc

N = 18432
K = 1024
D = 64
TN = 6144
GRID = N // TN
COMMITMENT_COST = 0.25
LOSS_SCALE = (1.0 + COMMITMENT_COST) / (N * D)

_SC_INFO = plsc.get_sparse_core_info()
NC = _SC_INFO.num_cores
NS = _SC_INFO.num_subcores
NW = NC * NS
BPW = N // NW
CHUNK = 96
NCHUNK = BPW // CHUNK


def _dist_argmin_body(zt_ref, wt_ref, idx_ref, loss_ref):
    i = pl.program_id(0)
    zt = zt_ref[...]
    wt = wt_ref[...]
    zsq = jnp.sum(zt * zt, axis=0, keepdims=True)
    wsq = jnp.sum(wt * wt, axis=0, keepdims=True)
    m = jax.lax.dot_general(wt, zt, (((0,), (0,)), ((), ())),
                            preferred_element_type=jnp.float32)
    d = (zsq + wsq.reshape(K, 1)) - 2.0 * m
    dmin = jnp.min(d, axis=0, keepdims=True)
    iota = jax.lax.broadcasted_iota(jnp.int32, d.shape, 0).astype(jnp.float32)
    idxf = jnp.min(jnp.where(d == dmin, iota, float(K)), axis=0)
    idx_ref[...] = idxf.astype(jnp.int32)[None, None, :]
    part = jnp.sum(dmin, keepdims=True)

    @pl.when(i == 0)
    def _init():
        loss_ref[...] = jnp.zeros_like(loss_ref)

    loss_ref[...] += part

    @pl.when(i == GRID - 1)
    def _final():
        loss_ref[...] = loss_ref[...] * LOSS_SCALE


_dist_argmin = pl.pallas_call(
    _dist_argmin_body,
    grid=(GRID,),
    in_specs=[
        pl.BlockSpec((D, TN), lambda i: (0, i)),
        pl.BlockSpec((D, K), lambda i: (0, 0)),
    ],
    out_specs=(
        pl.BlockSpec((1, 1, TN), lambda i: (i, 0, 0)),
        pl.BlockSpec((1, 1), lambda i: (0, 0)),
    ),
    out_shape=(
        jax.ShapeDtypeStruct((GRID, 1, TN), jnp.int32),
        jax.ShapeDtypeStruct((1, 1), jnp.float32),
    ),
)


@functools.partial(
    pl.kernel,
    mesh=plsc.VectorSubcoreMesh(core_axis_name="c", subcore_axis_name="s"),
    compiler_params=pltpu.CompilerParams(use_tc_tiling_on_sc=False),
    out_type=jax.ShapeDtypeStruct((N, D), jnp.float32),
    scratch_types=[
        pltpu.VMEM((NCHUNK, CHUNK), jnp.int32),
        pltpu.VMEM((BPW, D), jnp.float32),
        pltpu.SemaphoreType.DMA,
    ],
)
def _sc_gather(idx_hbm, w_hbm, out_hbm, idx_v, rows_v, sem):
    wid = lax.axis_index("s") * NC + lax.axis_index("c")
    pltpu.sync_copy(idx_hbm.at[wid], idx_v)
    copies = []
    for j in range(NCHUNK):
        copies.append(pltpu.async_copy(
            w_hbm.at[idx_v.at[j]],
            rows_v.at[pl.ds(j * CHUNK, CHUNK)],
            sem,
        ))
    for c in copies:
        c.wait()
    pltpu.sync_copy(rows_v, out_hbm.at[pl.ds(wid * BPW, BPW)])


def kernel(z, W):
    idx3, loss = _dist_argmin(z.T, W.T)
    idx = idx3.reshape(NW, NCHUNK, CHUNK)
    zq = _sc_gather(idx, W)
    return zq, loss[0, 0]
